# baseline (speedup 1.0000x reference)
import functools

import jax
import jax.numpy as jnp
from jax import lax
from jax.experimental import pallas as pl
from jax.experimental.pallas import tpu as pltpu
from jax.experimental.pallas import tpu_sc as plsc

_CHUNK = 128


@functools.cache
def _build_gather(B, V, D):
    info = plsc.get_sparse_core_info()
    nw = info.num_cores * info.num_subcores
    assert B % (nw * _CHUNK) == 0, (B, nw)
    b_per_w = B // nw
    n_chunks = b_per_w // _CHUNK

    mesh = plsc.VectorSubcoreMesh(core_axis_name="c", subcore_axis_name="s")

    @functools.partial(
        pl.kernel,
        mesh=mesh,
        out_type=jax.ShapeDtypeStruct((B, D), jnp.bfloat16),
        scratch_types=[
            pltpu.VMEM((n_chunks, _CHUNK), jnp.int32),
            pltpu.VMEM((b_per_w, D), jnp.bfloat16),
            pltpu.SemaphoreType.DMA,
        ],
        compiler_params=pltpu.CompilerParams(use_tc_tiling_on_sc=False),
    )
    def gather_kernel(table_hbm, idx_hbm, out_hbm, idx_v, rows_v, sem):
        wid = lax.axis_index("s") * info.num_cores + lax.axis_index("c")
        base = wid * b_per_w
        pltpu.sync_copy(idx_hbm.at[pl.ds(wid * n_chunks, n_chunks)], idx_v)
        copies = [
            pltpu.async_copy(
                table_hbm.at[idx_v.at[c]],
                rows_v.at[pl.ds(c * _CHUNK, _CHUNK)],
                sem,
            )
            for c in range(n_chunks)
        ]
        for cp in copies:
            cp.wait()
        pltpu.sync_copy(rows_v, out_hbm.at[pl.ds(base, b_per_w)])

    return gather_kernel


def kernel(weight, input_idx, dim):
    V, D = weight.shape
    B = input_idx.shape[0]
    idx = (input_idx + dim).astype(jnp.int32).reshape(-1, _CHUNK)
    out16 = _build_gather(B, V, D)(weight.astype(jnp.bfloat16), idx)
    return out16.astype(jnp.float32)


# speedup vs baseline: 1.2583x; 1.2583x over previous
import functools

import jax
import jax.numpy as jnp
from jax import lax
from jax.experimental import pallas as pl
from jax.experimental.pallas import tpu as pltpu
from jax.experimental.pallas import tpu_sc as plsc

_CHUNK = 128
_LANES = 16


@functools.cache
def _build_gather(B, V, D):
    info = plsc.get_sparse_core_info()
    nw = info.num_cores * info.num_subcores
    assert B % (nw * _CHUNK) == 0, (B, nw)
    assert 128 % D == 0
    pack = 128 // D
    b_per_w = B // nw
    n_chunks = b_per_w // _CHUNK

    mesh = plsc.VectorSubcoreMesh(core_axis_name="c", subcore_axis_name="s")

    @functools.partial(
        pl.kernel,
        mesh=mesh,
        out_type=jax.ShapeDtypeStruct((B // pack, 128), jnp.float32),
        scratch_types=[
            pltpu.VMEM((b_per_w,), jnp.int32),
            pltpu.VMEM((b_per_w,), jnp.int32),
            pltpu.VMEM((b_per_w, 128), jnp.float32),
            pltpu.VMEM((b_per_w // pack, 128), jnp.float32),
            pltpu.SemaphoreType.DMA,
        ],
        compiler_params=pltpu.CompilerParams(needs_layout_passes=False),
    )
    def gather_kernel(
        table_hbm, idxp_hbm, idx_hbm, out_hbm, idxp_v, idx_v, rows_v, out_v, sem
    ):
        wid = lax.axis_index("s") * info.num_cores + lax.axis_index("c")
        base = pl.multiple_of(wid * b_per_w, b_per_w)
        obase = pl.multiple_of(wid * (b_per_w // pack), b_per_w // pack)
        pltpu.sync_copy(idxp_hbm.at[pl.ds(base, b_per_w)], idxp_v)
        pltpu.sync_copy(idx_hbm.at[pl.ds(base, b_per_w)], idx_v)
        copies = [
            pltpu.async_copy(
                table_hbm.at[idxp_v.at[pl.ds(c * _CHUNK, _CHUNK)]],
                rows_v.at[pl.ds(c * _CHUNK, _CHUNK)],
                sem,
            )
            for c in range(n_chunks)
        ]
        for cp in copies:
            cp.wait()

        lane_iota = lax.iota(jnp.int32, _LANES)

        @pl.loop(0, b_per_w, step=_LANES)
        def extract(j_base):
            rows16 = j_base + lane_iota
            raw16 = idx_v[pl.ds(j_base, _LANES)]
            src_col0 = (lax.rem(raw16, pack)) * D
            dst_rows = lax.div(rows16, pack)
            dst_col0 = (lax.rem(rows16, pack)) * D
            for f in range(D):
                vals = plsc.load_gather(rows_v, [rows16, src_col0 + f])
                plsc.store_scatter(out_v, [dst_rows, dst_col0 + f], vals)

        pltpu.sync_copy(out_v, out_hbm.at[pl.ds(obase, b_per_w // pack)])

    return gather_kernel


def kernel(weight, input_idx, dim):
    V, D = weight.shape
    B = input_idx.shape[0]
    pack = 128 // D
    packed = weight.reshape(V // pack, pack * D)
    idx = (input_idx + dim).astype(jnp.int32)
    out_packed = _build_gather(B, V, D)(packed, idx // pack, idx)
    return out_packed.reshape(B, D)


# speedup vs baseline: 1.5053x; 1.1962x over previous
import functools

import jax
import jax.numpy as jnp
from jax import lax
from jax.experimental import pallas as pl
from jax.experimental.pallas import tpu as pltpu
from jax.experimental.pallas import tpu_sc as plsc

_CHUNK = 128


@functools.cache
def _build_gather(B, V, D):
    info = plsc.get_sparse_core_info()
    nw = info.num_cores * info.num_subcores
    assert B % (nw * _CHUNK) == 0, (B, nw)
    b_per_w = B // nw
    n_chunks = b_per_w // _CHUNK

    mesh = plsc.VectorSubcoreMesh(core_axis_name="c", subcore_axis_name="s")

    @functools.partial(
        pl.kernel,
        mesh=mesh,
        out_type=jax.ShapeDtypeStruct((B, 128), jnp.float32),
        scratch_types=[
            pltpu.VMEM((b_per_w,), jnp.int32),
            pltpu.VMEM((b_per_w, 128), jnp.float32),
            pltpu.SemaphoreType.DMA,
        ],
    )
    def gather_kernel(table_hbm, idx_hbm, out_hbm, idx_v, rows_v, sem):
        wid = lax.axis_index("s") * info.num_cores + lax.axis_index("c")
        base = pl.multiple_of(wid * b_per_w, b_per_w)
        pltpu.sync_copy(idx_hbm.at[pl.ds(base, b_per_w)], idx_v)
        copies = [
            pltpu.async_copy(
                table_hbm.at[idx_v.at[pl.ds(c * _CHUNK, _CHUNK)]],
                rows_v.at[pl.ds(c * _CHUNK, _CHUNK)],
                sem,
            )
            for c in range(n_chunks)
        ]
        for cp in copies:
            cp.wait()
        pltpu.sync_copy(rows_v, out_hbm.at[pl.ds(base, b_per_w)])

    return gather_kernel


def kernel(weight, input_idx, dim):
    V, D = weight.shape
    B = input_idx.shape[0]
    wide = jnp.pad(weight, ((0, 0), (0, 128 - D)))
    idx = (input_idx + dim).astype(jnp.int32)
    return _build_gather(B, V, D)(wide, idx)[:, :D]


# speedup vs baseline: 2.0353x; 1.3521x over previous
import functools

import jax
import jax.numpy as jnp
from jax import lax
from jax.experimental import pallas as pl
from jax.experimental.pallas import tpu as pltpu
from jax.experimental.pallas import tpu_sc as plsc

_LANES = 16


@functools.cache
def _build_gather(B, V, D):
    info = plsc.get_sparse_core_info()
    nw = info.num_cores * info.num_subcores
    assert B % (nw * _LANES) == 0, (B, nw)
    assert 128 % D == 0 and V % 8 == 0
    pack = 128 // D
    b_per_w = B // nw
    n_groups = b_per_w // _LANES

    mesh = plsc.VectorSubcoreMesh(core_axis_name="c", subcore_axis_name="s")

    @functools.partial(
        pl.kernel,
        mesh=mesh,
        out_type=jax.ShapeDtypeStruct((B // pack, 128), jnp.float32),
        scratch_types=[
            pltpu.VMEM((b_per_w,), jnp.int32),
            pltpu.VMEM((b_per_w,), jnp.int32),
            pltpu.VMEM((2, _LANES, 8, D), jnp.float32),
            pltpu.VMEM((b_per_w // pack, 128), jnp.float32),
            pltpu.SemaphoreType.DMA,
            pltpu.SemaphoreType.DMA,
        ],
        compiler_params=pltpu.CompilerParams(needs_layout_passes=False),
    )
    def gather_kernel(
        table_hbm,
        idxp_hbm,
        idx_hbm,
        out_hbm,
        idxp_v,
        idx_v,
        slab_v,
        out_v,
        sem0,
        sem1,
    ):
        wid = lax.axis_index("s") * info.num_cores + lax.axis_index("c")
        base = pl.multiple_of(wid * b_per_w, b_per_w)
        obase = pl.multiple_of(wid * (b_per_w // pack), b_per_w // pack)
        table3 = table_hbm.reshape(V // 8, 8, D)
        pltpu.sync_copy(idxp_hbm.at[pl.ds(base, b_per_w)], idxp_v)
        pltpu.sync_copy(idx_hbm.at[pl.ds(base, b_per_w)], idx_v)

        lane_iota = lax.iota(jnp.int32, _LANES)

        def fire(g, buf, sem):
            p16 = idxp_v[pl.ds(pl.multiple_of(g * _LANES, _LANES), _LANES)]
            for k in range(_LANES):
                pltpu.async_copy(
                    table3.at[p16[k]], slab_v.at[buf, k], sem
                )

        def drain(buf, sem):
            for _ in range(_LANES):
                pltpu.make_async_copy(
                    table3.at[0], slab_v.at[buf, 0], sem
                ).wait()

        def extract(g, buf):
            gbase = pl.multiple_of(g * _LANES, _LANES)
            raw16 = idx_v[pl.ds(gbase, _LANES)]
            j16 = lax.rem(raw16, 8)
            t16 = gbase + lane_iota
            or16 = lax.div(t16, pack)
            oc16 = lax.rem(t16, pack) * D

            @pl.loop(0, D)
            def per_feature(f):
                f16 = jnp.zeros((_LANES,), jnp.int32) + f
                vals = plsc.load_gather(slab_v.at[buf], [lane_iota, j16, f16])
                plsc.store_scatter(out_v, [or16, oc16 + f16], vals)

        assert n_groups % 2 == 0

        @pl.loop(0, n_groups, step=2)
        def per_pair(g):
            fire(g, 0, sem0)

            @pl.when(g >= 2)
            def _():
                drain(1, sem1)
                extract(g - 1, 1)

            fire(g + 1, 1, sem1)
            drain(0, sem0)
            extract(g, 0)

        drain(1, sem1)
        extract(n_groups - 1, 1)

        pltpu.sync_copy(out_v, out_hbm.at[pl.ds(obase, b_per_w // pack)])

    return gather_kernel


def kernel(weight, input_idx, dim):
    V, D = weight.shape
    B = input_idx.shape[0]
    pack = 128 // D
    idx = (input_idx + dim).astype(jnp.int32)
    out_packed = _build_gather(B, V, D)(weight, idx // 8, idx)
    return out_packed.reshape(B, D)


# speedup vs baseline: 2.8790x; 1.4146x over previous
import functools

import jax
import jax.numpy as jnp
from jax import lax
from jax.experimental import pallas as pl
from jax.experimental.pallas import tpu as pltpu
from jax.experimental.pallas import tpu_sc as plsc

_LANES = 16


@functools.cache
def _build_gather(B, V, D):
    info = plsc.get_sparse_core_info()
    nw = info.num_cores * info.num_subcores
    assert B % (nw * _LANES) == 0, (B, nw)
    assert 128 % D == 0 and V % 8 == 0
    pack = 128 // D
    b_per_w = B // nw
    n_groups = b_per_w // _LANES

    mesh = plsc.VectorSubcoreMesh(core_axis_name="c", subcore_axis_name="s")

    @functools.partial(
        pl.kernel,
        mesh=mesh,
        out_type=jax.ShapeDtypeStruct((B // pack, 128), jnp.float32),
        scratch_types=[
            pltpu.VMEM((b_per_w,), jnp.int32),
            pltpu.VMEM((b_per_w,), jnp.int32),
            pltpu.VMEM((2, _LANES, 8, D), jnp.float32),
            pltpu.VMEM((b_per_w // pack, 128), jnp.float32),
            pltpu.SemaphoreType.DMA,
            pltpu.SemaphoreType.DMA,
        ],
        compiler_params=pltpu.CompilerParams(needs_layout_passes=False),
    )
    def gather_kernel(
        table_hbm,
        idxp_hbm,
        idx_hbm,
        out_hbm,
        idxp_v,
        idx_v,
        slab_v,
        out_v,
        sem0,
        sem1,
    ):
        wid = lax.axis_index("s") * info.num_cores + lax.axis_index("c")
        base = pl.multiple_of(wid * b_per_w, b_per_w)
        obase = pl.multiple_of(wid * (b_per_w // pack), b_per_w // pack)
        table3 = table_hbm
        pltpu.sync_copy(idxp_hbm.at[pl.ds(base, b_per_w)], idxp_v)
        pltpu.sync_copy(idx_hbm.at[pl.ds(base, b_per_w)], idx_v)

        lane_iota = lax.iota(jnp.int32, _LANES)

        def fire(g, buf, sem):
            p16 = idxp_v[pl.ds(pl.multiple_of(g * _LANES, _LANES), _LANES)]
            for k in range(_LANES):
                pltpu.async_copy(
                    table3.at[p16[k]], slab_v.at[buf, k], sem
                )

        def drain(buf, sem):
            for _ in range(_LANES):
                pltpu.make_async_copy(
                    table3.at[0], slab_v.at[buf, 0], sem
                ).wait()

        def extract(g, buf):
            gbase = pl.multiple_of(g * _LANES, _LANES)
            raw16 = idx_v[pl.ds(gbase, _LANES)]
            j16 = lax.rem(raw16, 8)
            t16 = gbase + lane_iota
            or16 = lax.div(t16, pack)
            oc16 = lax.rem(t16, pack) * D
            zero16 = jnp.zeros((_LANES,), jnp.int32)

            @pl.loop(0, D, unroll=8)
            def per_feature(f):
                f16 = zero16 + f
                vals = plsc.load_gather(slab_v.at[buf], [lane_iota, j16, f16])
                plsc.store_scatter(out_v, [or16, oc16 + f16], vals)

        assert n_groups % 2 == 0

        @pl.loop(0, n_groups, step=2)
        def per_pair(g):
            fire(g, 0, sem0)

            @pl.when(g >= 2)
            def _():
                drain(1, sem1)
                extract(g - 1, 1)

            fire(g + 1, 1, sem1)
            drain(0, sem0)
            extract(g, 0)

        drain(1, sem1)
        extract(n_groups - 1, 1)

        pltpu.sync_copy(out_v, out_hbm.at[pl.ds(obase, b_per_w // pack)])

    return gather_kernel


def kernel(weight, input_idx, dim):
    V, D = weight.shape
    B = input_idx.shape[0]
    pack = 128 // D
    idx = (input_idx + dim).astype(jnp.int32)
    table3 = weight.reshape(V // 8, 8, D)
    out_packed = _build_gather(B, V, D)(table3, idx // 8, idx)
    return out_packed.reshape(B, D)


# speedup vs baseline: 3.1107x; 1.0805x over previous
import functools

import jax
import jax.numpy as jnp
from jax import lax
from jax.experimental import pallas as pl
from jax.experimental.pallas import tpu as pltpu
from jax.experimental.pallas import tpu_sc as plsc

_LANES = 16


@functools.cache
def _build_gather(B, V, D):
    info = plsc.get_sparse_core_info()
    nw = info.num_cores * info.num_subcores
    assert B % (nw * _LANES) == 0, (B, nw)
    assert 128 % D == 0 and V % 8 == 0
    pack = 128 // D
    b_per_w = B // nw
    n_groups = b_per_w // _LANES

    mesh = plsc.VectorSubcoreMesh(core_axis_name="c", subcore_axis_name="s")

    @functools.partial(
        pl.kernel,
        mesh=mesh,
        out_type=jax.ShapeDtypeStruct((D, B), jnp.float32),
        scratch_types=[
            pltpu.VMEM((b_per_w,), jnp.int32),
            pltpu.VMEM((b_per_w,), jnp.int32),
            pltpu.VMEM((2, _LANES, 8, D), jnp.float32),
            pltpu.VMEM((D, b_per_w), jnp.float32),
            pltpu.SemaphoreType.DMA,
            pltpu.SemaphoreType.DMA,
        ],
        compiler_params=pltpu.CompilerParams(needs_layout_passes=False),
    )
    def gather_kernel(
        table_hbm,
        idxp_hbm,
        idx_hbm,
        out_hbm,
        idxp_v,
        idx_v,
        slab_v,
        out_v,
        sem0,
        sem1,
    ):
        wid = lax.axis_index("s") * info.num_cores + lax.axis_index("c")
        base = pl.multiple_of(wid * b_per_w, b_per_w)
        obase = pl.multiple_of(wid * (b_per_w // pack), b_per_w // pack)
        table3 = table_hbm
        pltpu.sync_copy(idxp_hbm.at[pl.ds(base, b_per_w)], idxp_v)
        pltpu.sync_copy(idx_hbm.at[pl.ds(base, b_per_w)], idx_v)

        lane_iota = lax.iota(jnp.int32, _LANES)

        def fire(g, buf, sem):
            p16 = idxp_v[pl.ds(pl.multiple_of(g * _LANES, _LANES), _LANES)]
            for k in range(_LANES):
                pltpu.async_copy(
                    table3.at[p16[k]], slab_v.at[buf, k], sem
                )

        def drain(buf, sem):
            pltpu.make_async_copy(
                table3.at[pl.ds(0, _LANES)], slab_v.at[buf], sem
            ).wait()

        def extract(g, buf):
            gbase = pl.multiple_of(g * _LANES, _LANES)
            raw16 = idx_v[pl.ds(gbase, _LANES)]
            j16 = lax.rem(raw16, 8)
            t16 = gbase + lane_iota
            zero16 = jnp.zeros((_LANES,), jnp.int32)

            @pl.loop(0, D, unroll=8)
            def per_feature(f):
                f16 = zero16 + f
                vals = plsc.load_gather(slab_v.at[buf], [lane_iota, j16, f16])
                plsc.store_scatter(out_v, [f16, t16], vals)

        assert n_groups % 2 == 0

        @pl.loop(0, n_groups, step=2)
        def per_pair(g):
            fire(g, 0, sem0)

            @pl.when(g >= 2)
            def _():
                drain(1, sem1)
                extract(g - 1, 1)

            fire(g + 1, 1, sem1)
            drain(0, sem0)
            extract(g, 0)

        drain(1, sem1)
        extract(n_groups - 1, 1)

        pltpu.sync_copy(out_v, out_hbm.at[:, pl.ds(base, b_per_w)])

    return gather_kernel


def kernel(weight, input_idx, dim):
    V, D = weight.shape
    B = input_idx.shape[0]
    pack = 128 // D
    idx = (input_idx + dim).astype(jnp.int32)
    table3 = weight.reshape(V // 8, 8, D)
    outT = _build_gather(B, V, D)(table3, idx // 8, idx)
    return outT.T


# speedup vs baseline: 3.1365x; 1.0083x over previous
import functools

import jax
import jax.numpy as jnp
from jax import lax
from jax.experimental import pallas as pl
from jax.experimental.pallas import tpu as pltpu
from jax.experimental.pallas import tpu_sc as plsc

_LANES = 16
_GROUP = 32


@functools.cache
def _build_gather(B, V, D):
    info = plsc.get_sparse_core_info()
    nw = info.num_cores * info.num_subcores
    assert B % (nw * _GROUP) == 0, (B, nw)
    assert 128 % D == 0 and V % 8 == 0
    b_per_w = B // nw
    n_groups = b_per_w // _GROUP

    mesh = plsc.VectorSubcoreMesh(core_axis_name="c", subcore_axis_name="s")

    @functools.partial(
        pl.kernel,
        mesh=mesh,
        out_type=jax.ShapeDtypeStruct((D, B), jnp.float32),
        scratch_types=[
            pltpu.VMEM((b_per_w,), jnp.int32),
            pltpu.VMEM((b_per_w,), jnp.int32),
            pltpu.VMEM((2, _GROUP, 8, D), jnp.float32),
            pltpu.VMEM((D, b_per_w), jnp.float32),
            pltpu.SemaphoreType.DMA,
            pltpu.SemaphoreType.DMA,
        ],
        compiler_params=pltpu.CompilerParams(needs_layout_passes=False),
    )
    def gather_kernel(
        table_hbm,
        idxp_hbm,
        idx_hbm,
        out_hbm,
        idxp_v,
        idx_v,
        slab_v,
        out_v,
        sem0,
        sem1,
    ):
        wid = lax.axis_index("s") * info.num_cores + lax.axis_index("c")
        base = pl.multiple_of(wid * b_per_w, b_per_w)
        table3 = table_hbm
        pltpu.sync_copy(idxp_hbm.at[pl.ds(base, b_per_w)], idxp_v)
        pltpu.sync_copy(idx_hbm.at[pl.ds(base, b_per_w)], idx_v)

        lane_iota = lax.iota(jnp.int32, _LANES)

        def fire(g, buf, sem):
            for h in range(_GROUP // _LANES):
                p16 = idxp_v[
                    pl.ds(pl.multiple_of(g * _GROUP + h * _LANES, _LANES), _LANES)
                ]
                for k in range(_LANES):
                    pltpu.async_copy(
                        table3.at[p16[k]], slab_v.at[buf, h * _LANES + k], sem
                    )

        def drain(buf, sem):
            pltpu.make_async_copy(
                table3.at[pl.ds(0, _GROUP)], slab_v.at[buf], sem
            ).wait()

        def extract(g, buf):
            zero16 = jnp.zeros((_LANES,), jnp.int32)
            for h in range(_GROUP // _LANES):
                gbase = pl.multiple_of(g * _GROUP + h * _LANES, _LANES)
                raw16 = idx_v[pl.ds(gbase, _LANES)]
                j16 = lax.rem(raw16, 8)
                t16 = gbase + lane_iota
                s16 = h * _LANES + lane_iota

                @pl.loop(0, D, unroll=8)
                def per_feature(f):
                    f16 = zero16 + f
                    vals = plsc.load_gather(slab_v.at[buf], [s16, j16, f16])
                    plsc.store_scatter(out_v, [f16, t16], vals)

        assert n_groups % 2 == 0

        @pl.loop(0, n_groups, step=2)
        def per_pair(g):
            fire(g, 0, sem0)

            @pl.when(g >= 2)
            def _():
                drain(1, sem1)
                extract(g - 1, 1)

            fire(g + 1, 1, sem1)
            drain(0, sem0)
            extract(g, 0)

        drain(1, sem1)
        extract(n_groups - 1, 1)

        pltpu.sync_copy(out_v, out_hbm.at[:, pl.ds(base, b_per_w)])

    return gather_kernel


def kernel(weight, input_idx, dim):
    V, D = weight.shape
    B = input_idx.shape[0]
    idx = (input_idx + dim).astype(jnp.int32)
    table3 = weight.reshape(V // 8, 8, D)
    outT = _build_gather(B, V, D)(table3, idx // 8, idx)
    return outT.T
